# baseline (speedup 1.0000x reference)
import functools

import jax
import jax.numpy as jnp
from jax import lax
from jax.experimental import pallas as pl
from jax.experimental.pallas import tpu as pltpu
from jax.experimental.pallas import tpu_sc as plsc

NC, NS = 2, 16
NW = NC * NS
CHUNK = 80
LANES = 16


def _sc_segsum(xpad, src, dst, zeros, n):
  n2, d = xpad.shape
  nit = src.shape[1]
  ndeg = (n2 + 2 * LANES - 1) // LANES * LANES
  rpt = (n // NS) // 8 * 8
  tail = n - (NS - 1) * rpt
  rpt2 = (n2 // NS) // 8 * 8
  tail2 = n2 - (NS - 1) * rpt2

  mesh = plsc.VectorSubcoreMesh(core_axis_name="c", subcore_axis_name="s")

  @functools.partial(
      pl.kernel,
      out_type=(jax.ShapeDtypeStruct((NC * n, d), jnp.float32),
                jax.ShapeDtypeStruct((NW, ndeg), jnp.float32)),
      mesh=mesh,
      compiler_params=pltpu.CompilerParams(needs_layout_passes=False),
      scratch_types=[
          pltpu.VMEM_SHARED((n2, d), jnp.float32),
          pltpu.VMEM((nit // 2, CHUNK), jnp.int32),
          pltpu.VMEM((nit // 2, CHUNK), jnp.int32),
          pltpu.VMEM((2, CHUNK, d), jnp.float32),
          pltpu.VMEM((ndeg,), jnp.float32),
          pltpu.SemaphoreType.DMA,
          pltpu.SemaphoreType.DMA,
      ],
  )
  def seg_kernel(x_hbm, src_hbm, dst_hbm, zeros_hbm, out_hbm, deg_hbm,
                 acc, src_v, dst_v, rows_v, deg_v, gsem0, gsem1):
    cid = lax.axis_index("c")
    sid = lax.axis_index("s")
    wid = cid * NS + sid
    gsems = (gsem0, gsem1)
    half = nit // 2

    z0 = sid * rpt2
    pltpu.sync_copy(zeros_hbm.at[pl.ds(z0, rpt2)], acc.at[pl.ds(z0, rpt2)])

    @pl.when(sid == NS - 1)
    def _zero_tail():
      t0 = NS * rpt2
      pltpu.sync_copy(zeros_hbm.at[pl.ds(t0, tail2 - rpt2)],
                      acc.at[pl.ds(t0, tail2 - rpt2)])

    def zbody(i, _):
      deg_v[pl.ds(i * LANES, LANES)] = jnp.zeros((LANES,), jnp.float32)
      return _

    lax.fori_loop(0, ndeg // LANES, zbody, 0, unroll=False)
    plsc.subcore_barrier()

    def gather_start(l, p):
      pltpu.async_copy(x_hbm.at[src_v.at[l]], rows_v.at[p], gsems[p])

    def gather_wait(l, p):
      pltpu.make_async_copy(x_hbm.at[src_v.at[l]], rows_v.at[p],
                            gsems[p]).wait()

    def step(l, p, prefetch):
      gather_wait(l, p)
      if prefetch:
        gather_start(l + 1, 1 - p)
      for j in range(CHUNK // LANES):
        dvec = dst_v[l, pl.ds(j * LANES, LANES)]
        cnt, last = plsc.scan_count(dvec)
        plsc.addupdate_scatter(deg_v, [dvec], cnt.astype(jnp.float32),
                               mask=last)
      pltpu.sync_copy(rows_v.at[p], acc.at[dst_v.at[l]], add=True)

    for base in (0, half):
      pltpu.sync_copy(src_hbm.at[wid, pl.ds(base, half)], src_v)
      pltpu.sync_copy(dst_hbm.at[wid, pl.ds(base, half)], dst_v)
      gather_start(0, 0)

      def body(o, _):
        step(2 * o, 0, True)
        step(2 * o + 1, 1, True)
        return _

      lax.fori_loop(0, half // 2 - 1, body, 0, unroll=False)
      step(half - 2, 0, True)
      step(half - 1, 1, False)
    plsc.subcore_barrier()

    r0 = sid * rpt
    pltpu.sync_copy(acc.at[pl.ds(r0, rpt)],
                    out_hbm.at[pl.ds(cid * n + r0, rpt)])

    @pl.when(sid == NS - 1)
    def _write_tail():
      t0 = NS * rpt
      pltpu.sync_copy(acc.at[pl.ds(t0, tail - rpt)],
                      out_hbm.at[pl.ds(cid * n + t0, tail - rpt)])

    pltpu.sync_copy(deg_v, deg_hbm.at[wid])

  return seg_kernel(xpad, src, dst, zeros)


def _tc_mlp_body(x_ref, a0_ref, a1_ref, deg_ref, w1a_ref, w1b_ref, b1_ref,
                 wc1_ref, bc1_ref, pw_ref, wc2_ref, bc2_ref, out_ref):
  a = a0_ref[...] + a1_ref[...]
  deg = jnp.sum(deg_ref[...], axis=1)[:, None]
  agg = a / jnp.maximum(deg, 1.0)
  h = x_ref[...] @ w1a_ref[...] + agg @ w1b_ref[...] + b1_ref[...][None, :]
  h = jnp.maximum(h, 0.0)
  p1 = h @ wc1_ref[...] + bc1_ref[...][None, :]
  p1 = jnp.where(p1 >= 0, p1, pw_ref[...][None, :] * p1)
  out_ref[...] = p1 @ wc2_ref[...] + bc2_ref[...][None, :]


def _tc_mlp(x, partials, degs, w1a, w1b, b1, wc1, bc1, prelu_w, wc2, bc2):
  n, d = x.shape
  h = wc1.shape[0]
  c = wc2.shape[1]
  bn = 1000
  grid = n // bn

  full = lambda shape: pl.BlockSpec(shape, lambda i: (0,) * len(shape))
  return pl.pallas_call(
      _tc_mlp_body,
      grid=(grid,),
      in_specs=[
          pl.BlockSpec((bn, d), lambda i: (i, 0)),
          pl.BlockSpec((bn, d), lambda i: (i, 0)),
          pl.BlockSpec((bn, d), lambda i: (i + n // bn, 0)),
          pl.BlockSpec((bn, NW), lambda i: (i, 0)),
          full((d, h)), full((d, h)), full((h,)),
          full((h, h)), full((h,)), full((h,)),
          full((h, c)), full((c,)),
      ],
      out_specs=pl.BlockSpec((bn, c), lambda i: (i, 0)),
      out_shape=jax.ShapeDtypeStruct((n, c), jnp.float32),
  )(x, partials, partials, degs, w1a, w1b, b1, wc1, bc1, prelu_w, wc2, bc2)


def kernel(x, edge_index, labels, W1, b1, Wc1, bc1, prelu_w, Wc2, bc2):
  n, d = x.shape
  e = edge_index.shape[1]
  nit = (-(-e // (NW * CHUNK)) + 15) // 16 * 16
  e_pad = NW * nit * CHUNK
  n2 = n + 8
  xpad = jnp.zeros((n2, d), jnp.float32).at[:n].set(x)
  zeros = jnp.zeros((n2, d), jnp.float32)
  ghosts = jnp.full((e_pad - e,), n, dtype=edge_index.dtype)
  src = jnp.concatenate([edge_index[0], ghosts]).reshape(NW, nit, CHUNK)
  dst = jnp.concatenate([edge_index[1], ghosts]).reshape(NW, nit, CHUNK)
  partials, degs = _sc_segsum(xpad, src, dst, zeros, n)
  return _tc_mlp(x, partials, degs.T, W1[:d], W1[d:], b1,
                 Wc1, bc1, prelu_w, Wc2, bc2)


# speedup vs baseline: 2.1179x; 2.1179x over previous
import functools

import jax
import jax.numpy as jnp
from jax import lax
from jax.experimental import pallas as pl
from jax.experimental.pallas import tpu as pltpu
from jax.experimental.pallas import tpu_sc as plsc

NC, NS = 2, 16
NW = NC * NS
CHUNK = 80
LANES = 16


def _sc_segsum(xpad, src, dst, zeros, n):
  n2, d = xpad.shape
  nit = src.shape[1]
  ndeg = (n2 + 2 * LANES - 1) // LANES * LANES
  rpt = (n // NS) // 8 * 8
  tail = n - (NS - 1) * rpt
  rpt2 = (n2 // NS) // 8 * 8
  tail2 = n2 - (NS - 1) * rpt2

  mesh = plsc.VectorSubcoreMesh(core_axis_name="c", subcore_axis_name="s")

  @functools.partial(
      pl.kernel,
      out_type=(jax.ShapeDtypeStruct((NC * n, d), jnp.float32),
                jax.ShapeDtypeStruct((NW, ndeg), jnp.float32)),
      mesh=mesh,
      compiler_params=pltpu.CompilerParams(needs_layout_passes=False),
      scratch_types=[
          pltpu.VMEM_SHARED((n2, d), jnp.float32),
          pltpu.VMEM((nit // 2, CHUNK), jnp.int32),
          pltpu.VMEM((nit // 2, CHUNK), jnp.int32),
          pltpu.VMEM((2, CHUNK, d), jnp.float32),
          pltpu.VMEM((ndeg,), jnp.float32),
          pltpu.SemaphoreType.DMA,
          pltpu.SemaphoreType.DMA,
      ],
  )
  def seg_kernel(x_hbm, src_hbm, dst_hbm, zeros_hbm, out_hbm, deg_hbm,
                 acc, src_v, dst_v, rows_v, deg_v, gsem0, gsem1):
    cid = lax.axis_index("c")
    sid = lax.axis_index("s")
    wid = cid * NS + sid
    gsems = (gsem0, gsem1)
    half = nit // 2

    z0 = sid * rpt2
    pltpu.sync_copy(zeros_hbm.at[pl.ds(z0, rpt2)], acc.at[pl.ds(z0, rpt2)])

    @pl.when(sid == NS - 1)
    def _zero_tail():
      t0 = NS * rpt2
      pltpu.sync_copy(zeros_hbm.at[pl.ds(t0, tail2 - rpt2)],
                      acc.at[pl.ds(t0, tail2 - rpt2)])

    def zbody(i, _):
      deg_v[pl.ds(i * LANES, LANES)] = jnp.zeros((LANES,), jnp.float32)
      return _

    lax.fori_loop(0, ndeg // LANES, zbody, 0, unroll=False)
    plsc.subcore_barrier()

    def gather_start(l, p):
      pltpu.async_copy(x_hbm.at[src_v.at[l]], rows_v.at[p], gsems[p])

    def gather_wait(l, p):
      pltpu.make_async_copy(x_hbm.at[src_v.at[l]], rows_v.at[p],
                            gsems[p]).wait()

    def step(l, p, prefetch):
      gather_wait(l, p)
      if prefetch:
        gather_start(l + 1, 1 - p)
      for j in range(CHUNK // LANES):
        dvec = dst_v[l, pl.ds(j * LANES, LANES)]
        cnt, last = plsc.scan_count(dvec)
        plsc.addupdate_scatter(deg_v, [dvec], cnt.astype(jnp.float32),
                               mask=last)
      pltpu.sync_copy(rows_v.at[p], acc.at[dst_v.at[l]], add=True)

    for base in (0, half):
      pltpu.sync_copy(src_hbm.at[wid, pl.ds(base, half)], src_v)
      pltpu.sync_copy(dst_hbm.at[wid, pl.ds(base, half)], dst_v)
      gather_start(0, 0)

      def body(o, _):
        step(2 * o, 0, True)
        step(2 * o + 1, 1, True)
        return _

      lax.fori_loop(0, half // 2 - 1, body, 0, unroll=False)
      step(half - 2, 0, True)
      step(half - 1, 1, False)
    plsc.subcore_barrier()

    r0 = sid * rpt
    pltpu.sync_copy(acc.at[pl.ds(r0, rpt)],
                    out_hbm.at[pl.ds(cid * n + r0, rpt)])

    @pl.when(sid == NS - 1)
    def _write_tail():
      t0 = NS * rpt
      pltpu.sync_copy(acc.at[pl.ds(t0, tail - rpt)],
                      out_hbm.at[pl.ds(cid * n + t0, tail - rpt)])

    pltpu.sync_copy(deg_v, deg_hbm.at[wid])

  return seg_kernel(xpad, src, dst, zeros)


def _tc_mlp_body(x_ref, a0_ref, a1_ref, deg_ref, w1a_ref, w1b_ref, b1_ref,
                 wc1_ref, bc1_ref, pw_ref, wc2_ref, bc2_ref, out_ref):
  a = a0_ref[...] + a1_ref[...]
  deg = jnp.sum(deg_ref[...], axis=1)[:, None]
  agg = a / jnp.maximum(deg, 1.0)
  h = x_ref[...] @ w1a_ref[...] + agg @ w1b_ref[...] + b1_ref[...][None, :]
  h = jnp.maximum(h, 0.0)
  p1 = h @ wc1_ref[...] + bc1_ref[...][None, :]
  p1 = jnp.where(p1 >= 0, p1, pw_ref[...][None, :] * p1)
  out_ref[...] = p1 @ wc2_ref[...] + bc2_ref[...][None, :]


def _tc_mlp(x, partials, degs, w1a, w1b, b1, wc1, bc1, prelu_w, wc2, bc2):
  n, d = x.shape
  h = wc1.shape[0]
  c = wc2.shape[1]
  bn = 1000
  grid = n // bn

  full = lambda shape: pl.BlockSpec(shape, lambda i: (0,) * len(shape))
  return pl.pallas_call(
      _tc_mlp_body,
      grid=(grid,),
      in_specs=[
          pl.BlockSpec((bn, d), lambda i: (i, 0)),
          pl.BlockSpec((bn, d), lambda i: (i, 0)),
          pl.BlockSpec((bn, d), lambda i: (i + n // bn, 0)),
          pl.BlockSpec((bn, NW), lambda i: (i, 0)),
          full((d, h)), full((d, h)), full((h,)),
          full((h, h)), full((h,)), full((h,)),
          full((h, c)), full((c,)),
      ],
      out_specs=pl.BlockSpec((bn, c), lambda i: (i, 0)),
      out_shape=jax.ShapeDtypeStruct((n, c), jnp.float32),
  )(x, partials, partials, degs, w1a, w1b, b1, wc1, bc1, prelu_w, wc2, bc2)


def kernel(x, edge_index, labels, W1, b1, Wc1, bc1, prelu_w, Wc2, bc2):
  n, d = x.shape
  e = edge_index.shape[1]
  nit = (-(-e // (NW * CHUNK)) + 15) // 16 * 16
  per_tile = e // NW
  pad_per_tile = nit * CHUNK - per_tile
  n2 = n + NS
  xpad = jnp.zeros((n2, d), jnp.float32).at[:n].set(x)
  zeros = jnp.zeros((n2, d), jnp.float32)
  ghosts = jnp.broadcast_to(
      (n + jnp.arange(NW, dtype=edge_index.dtype) % NS)[:, None],
      (NW, pad_per_tile))
  src = jnp.concatenate(
      [edge_index[0].reshape(NW, per_tile), ghosts], axis=1
  ).reshape(NW, nit, CHUNK)
  dst = jnp.concatenate(
      [edge_index[1].reshape(NW, per_tile), ghosts], axis=1
  ).reshape(NW, nit, CHUNK)
  partials, degs = _sc_segsum(xpad, src, dst, zeros, n)
  return _tc_mlp(x, partials, degs.T, W1[:d], W1[d:], b1,
                 Wc1, bc1, prelu_w, Wc2, bc2)


# speedup vs baseline: 2.5270x; 1.1931x over previous
import functools

import jax
import jax.numpy as jnp
from jax import lax
from jax.experimental import pallas as pl
from jax.experimental.pallas import tpu as pltpu
from jax.experimental.pallas import tpu_sc as plsc

NC, NS = 2, 16
NW = NC * NS
CHUNK = 80
LANES = 16


def _sc_segsum(xpad, src, dst, zeros, n):
  n2, d = xpad.shape
  nit = src.shape[1]
  ndeg = (n2 + 2 * LANES - 1) // LANES * LANES
  rpt = (n // NS) // 8 * 8
  tail = n - (NS - 1) * rpt
  rpt2 = (n2 // NS) // 8 * 8
  tail2 = n2 - (NS - 1) * rpt2

  mesh = plsc.VectorSubcoreMesh(core_axis_name="c", subcore_axis_name="s")

  @functools.partial(
      pl.kernel,
      out_type=(jax.ShapeDtypeStruct((NC * n, d), jnp.float32),
                jax.ShapeDtypeStruct((NW, ndeg), jnp.float32)),
      mesh=mesh,
      compiler_params=pltpu.CompilerParams(needs_layout_passes=False),
      scratch_types=[
          pltpu.VMEM_SHARED((n2, d), jnp.float32),
          pltpu.VMEM((nit // 2, CHUNK), jnp.int32),
          pltpu.VMEM((nit // 2, CHUNK), jnp.int32),
          pltpu.VMEM((2, CHUNK, d), jnp.float32),
          pltpu.VMEM((ndeg,), jnp.float32),
          pltpu.SemaphoreType.DMA,
          pltpu.SemaphoreType.DMA,
          pltpu.SemaphoreType.DMA,
          pltpu.SemaphoreType.DMA,
      ],
  )
  def seg_kernel(x_hbm, src_hbm, dst_hbm, zeros_hbm, out_hbm, deg_hbm,
                 acc, src_v, dst_v, rows_v, deg_v, gsem0, gsem1, ssem0, ssem1):
    cid = lax.axis_index("c")
    sid = lax.axis_index("s")
    wid = cid * NS + sid
    gsems = (gsem0, gsem1)
    ssems = (ssem0, ssem1)
    half = nit // 2

    z0 = sid * rpt2
    pltpu.sync_copy(zeros_hbm.at[pl.ds(z0, rpt2)], acc.at[pl.ds(z0, rpt2)])

    @pl.when(sid == NS - 1)
    def _zero_tail():
      t0 = NS * rpt2
      pltpu.sync_copy(zeros_hbm.at[pl.ds(t0, tail2 - rpt2)],
                      acc.at[pl.ds(t0, tail2 - rpt2)])

    def zbody(i, _):
      deg_v[pl.ds(i * LANES, LANES)] = jnp.zeros((LANES,), jnp.float32)
      return _

    lax.fori_loop(0, ndeg // LANES, zbody, 0, unroll=False)
    plsc.subcore_barrier()

    def gather_start(l, p):
      pltpu.async_copy(x_hbm.at[src_v.at[l]], rows_v.at[p], gsems[p])

    def gather_wait(l, p):
      pltpu.make_async_copy(x_hbm.at[src_v.at[l]], rows_v.at[p],
                            gsems[p]).wait()

    def scatter_start(l, p):
      pltpu.async_copy(rows_v.at[p], acc.at[dst_v.at[l]], ssems[p], add=True)

    def scatter_wait(l, p):
      pltpu.make_async_copy(rows_v.at[p], acc.at[dst_v.at[l]],
                            ssems[p]).wait()

    def degrees(l):
      for j in range(CHUNK // LANES):
        dvec = dst_v[l, pl.ds(j * LANES, LANES)]
        cnt, last = plsc.scan_count(dvec)
        plsc.addupdate_scatter(deg_v, [dvec], cnt.astype(jnp.float32),
                               mask=last)

    def step_mid(l, p):
      scatter_wait(l - 1, 1 - p)
      gather_start(l + 1, 1 - p)
      gather_wait(l, p)
      scatter_start(l, p)
      degrees(l)

    for base in (0, half):
      pltpu.sync_copy(src_hbm.at[wid, pl.ds(base, half)], src_v)
      pltpu.sync_copy(dst_hbm.at[wid, pl.ds(base, half)], dst_v)
      gather_start(0, 0)
      gather_wait(0, 0)
      gather_start(1, 1)
      scatter_start(0, 0)
      degrees(0)

      def body(o, _):
        step_mid(2 * o + 1, 1)
        step_mid(2 * o + 2, 0)
        return _

      lax.fori_loop(0, half // 2 - 1, body, 0, unroll=False)
      scatter_wait(half - 2, 0)
      gather_wait(half - 1, 1)
      scatter_start(half - 1, 1)
      degrees(half - 1)
      scatter_wait(half - 1, 1)
    plsc.subcore_barrier()

    r0 = sid * rpt
    pltpu.sync_copy(acc.at[pl.ds(r0, rpt)],
                    out_hbm.at[pl.ds(cid * n + r0, rpt)])

    @pl.when(sid == NS - 1)
    def _write_tail():
      t0 = NS * rpt
      pltpu.sync_copy(acc.at[pl.ds(t0, tail - rpt)],
                      out_hbm.at[pl.ds(cid * n + t0, tail - rpt)])

    pltpu.sync_copy(deg_v, deg_hbm.at[wid])

  return seg_kernel(xpad, src, dst, zeros)


def _tc_mlp_body(x_ref, a0_ref, a1_ref, deg_ref, w1a_ref, w1b_ref, b1_ref,
                 wc1_ref, bc1_ref, pw_ref, wc2_ref, bc2_ref, out_ref):
  a = a0_ref[...] + a1_ref[...]
  deg = jnp.sum(deg_ref[...], axis=1)[:, None]
  agg = a / jnp.maximum(deg, 1.0)
  h = x_ref[...] @ w1a_ref[...] + agg @ w1b_ref[...] + b1_ref[...][None, :]
  h = jnp.maximum(h, 0.0)
  p1 = h @ wc1_ref[...] + bc1_ref[...][None, :]
  p1 = jnp.where(p1 >= 0, p1, pw_ref[...][None, :] * p1)
  out_ref[...] = p1 @ wc2_ref[...] + bc2_ref[...][None, :]


def _tc_mlp(x, partials, degs, w1a, w1b, b1, wc1, bc1, prelu_w, wc2, bc2):
  n, d = x.shape
  h = wc1.shape[0]
  c = wc2.shape[1]
  bn = 1000
  grid = n // bn

  full = lambda shape: pl.BlockSpec(shape, lambda i: (0,) * len(shape))
  return pl.pallas_call(
      _tc_mlp_body,
      grid=(grid,),
      in_specs=[
          pl.BlockSpec((bn, d), lambda i: (i, 0)),
          pl.BlockSpec((bn, d), lambda i: (i, 0)),
          pl.BlockSpec((bn, d), lambda i: (i + n // bn, 0)),
          pl.BlockSpec((bn, NW), lambda i: (i, 0)),
          full((d, h)), full((d, h)), full((h,)),
          full((h, h)), full((h,)), full((h,)),
          full((h, c)), full((c,)),
      ],
      out_specs=pl.BlockSpec((bn, c), lambda i: (i, 0)),
      out_shape=jax.ShapeDtypeStruct((n, c), jnp.float32),
  )(x, partials, partials, degs, w1a, w1b, b1, wc1, bc1, prelu_w, wc2, bc2)


def kernel(x, edge_index, labels, W1, b1, Wc1, bc1, prelu_w, Wc2, bc2):
  n, d = x.shape
  e = edge_index.shape[1]
  nit = (-(-e // (NW * CHUNK)) + 15) // 16 * 16
  per_tile = e // NW
  pad_per_tile = nit * CHUNK - per_tile
  n2 = n + NS
  xpad = jnp.zeros((n2, d), jnp.float32).at[:n].set(x)
  zeros = jnp.zeros((n2, d), jnp.float32)
  ghosts = jnp.broadcast_to(
      (n + jnp.arange(NW, dtype=edge_index.dtype) % NS)[:, None],
      (NW, pad_per_tile))
  src = jnp.concatenate(
      [edge_index[0].reshape(NW, per_tile), ghosts], axis=1
  ).reshape(NW, nit, CHUNK)
  dst = jnp.concatenate(
      [edge_index[1].reshape(NW, per_tile), ghosts], axis=1
  ).reshape(NW, nit, CHUNK)
  partials, degs = _sc_segsum(xpad, src, dst, zeros, n)
  return _tc_mlp(x, partials, degs.T, W1[:d], W1[d:], b1,
                 Wc1, bc1, prelu_w, Wc2, bc2)


# speedup vs baseline: 2.6575x; 1.0516x over previous
import functools

import jax
import jax.numpy as jnp
from jax import lax
from jax.experimental import pallas as pl
from jax.experimental.pallas import tpu as pltpu
from jax.experimental.pallas import tpu_sc as plsc

NC, NS = 2, 16
NW = NC * NS
CHUNK = 80
LANES = 16


def _sc_segsum(x, src, dst, zeros, n2):
  n, d = x.shape
  nit = src.shape[1]
  ndeg = (n2 + 2 * LANES - 1) // LANES * LANES
  rpt = (n // NS) // 8 * 8
  tail = n - (NS - 1) * rpt
  rpt2 = (n2 // NS) // 8 * 8
  tail2 = n2 - (NS - 1) * rpt2

  mesh = plsc.VectorSubcoreMesh(core_axis_name="c", subcore_axis_name="s")

  @functools.partial(
      pl.kernel,
      out_type=(jax.ShapeDtypeStruct((NC, n, d), jnp.float32),
                jax.ShapeDtypeStruct((NW, ndeg), jnp.float32)),
      mesh=mesh,
      compiler_params=pltpu.CompilerParams(needs_layout_passes=False),
      scratch_types=[
          pltpu.VMEM_SHARED((n2, d), jnp.float32),
          pltpu.VMEM((nit // 2, CHUNK), jnp.int32),
          pltpu.VMEM((nit // 2, CHUNK), jnp.int32),
          pltpu.VMEM((2, CHUNK, d), jnp.float32),
          pltpu.VMEM((ndeg,), jnp.float32),
          pltpu.SemaphoreType.DMA,
          pltpu.SemaphoreType.DMA,
          pltpu.SemaphoreType.DMA,
          pltpu.SemaphoreType.DMA,
      ],
  )
  def seg_kernel(x_hbm, src_hbm, dst_hbm, zeros_hbm, out_hbm, deg_hbm,
                 acc, src_v, dst_v, rows_v, deg_v, gsem0, gsem1, ssem0, ssem1):
    cid = lax.axis_index("c")
    sid = lax.axis_index("s")
    wid = cid * NS + sid
    gsems = (gsem0, gsem1)
    ssems = (ssem0, ssem1)
    half = nit // 2

    z0 = sid * rpt2
    pltpu.sync_copy(zeros_hbm.at[pl.ds(0, rpt2)], acc.at[pl.ds(z0, rpt2)])

    @pl.when(sid == NS - 1)
    def _zero_tail():
      t0 = NS * rpt2
      pltpu.sync_copy(zeros_hbm.at[pl.ds(0, tail2 - rpt2)],
                      acc.at[pl.ds(t0, tail2 - rpt2)])

    def zbody(i, _):
      deg_v[pl.ds(i * LANES, LANES)] = jnp.zeros((LANES,), jnp.float32)
      return _

    lax.fori_loop(0, ndeg // LANES, zbody, 0, unroll=False)
    plsc.subcore_barrier()

    def gather_start(l, p):
      pltpu.async_copy(x_hbm.at[src_v.at[l]], rows_v.at[p], gsems[p])

    def gather_wait(l, p):
      pltpu.make_async_copy(x_hbm.at[src_v.at[l]], rows_v.at[p],
                            gsems[p]).wait()

    def scatter_start(l, p):
      pltpu.async_copy(rows_v.at[p], acc.at[dst_v.at[l]], ssems[p], add=True)

    def scatter_wait(l, p):
      pltpu.make_async_copy(rows_v.at[p], acc.at[dst_v.at[l]],
                            ssems[p]).wait()

    def degrees(l):
      for j in range(CHUNK // LANES):
        dvec = dst_v[l, pl.ds(j * LANES, LANES)]
        cnt, last = plsc.scan_count(dvec)
        plsc.addupdate_scatter(deg_v, [dvec], cnt.astype(jnp.float32),
                               mask=last)

    def step_mid(l, p):
      scatter_wait(l - 1, 1 - p)
      gather_start(l + 1, 1 - p)
      gather_wait(l, p)
      scatter_start(l, p)
      degrees(l)

    for base in (0, half):
      pltpu.sync_copy(src_hbm.at[wid, pl.ds(base, half)], src_v)
      pltpu.sync_copy(dst_hbm.at[wid, pl.ds(base, half)], dst_v)
      gather_start(0, 0)
      gather_wait(0, 0)
      gather_start(1, 1)
      scatter_start(0, 0)
      degrees(0)

      def body(o, _):
        step_mid(2 * o + 1, 1)
        step_mid(2 * o + 2, 0)
        return _

      lax.fori_loop(0, half // 2 - 1, body, 0, unroll=False)
      scatter_wait(half - 2, 0)
      gather_wait(half - 1, 1)
      scatter_start(half - 1, 1)
      degrees(half - 1)
      scatter_wait(half - 1, 1)
    plsc.subcore_barrier()

    r0 = sid * rpt
    pltpu.sync_copy(acc.at[pl.ds(r0, rpt)],
                    out_hbm.at[cid, pl.ds(r0, rpt)])

    @pl.when(sid == NS - 1)
    def _write_tail():
      t0 = NS * rpt
      pltpu.sync_copy(acc.at[pl.ds(t0, tail - rpt)],
                      out_hbm.at[cid, pl.ds(t0, tail - rpt)])

    pltpu.sync_copy(deg_v, deg_hbm.at[wid])

  return seg_kernel(x, src, dst, zeros)


def _tc_mlp_body(x_ref, a0_ref, a1_ref, deg_ref, w1a_ref, w1b_ref, b1_ref,
                 wc1_ref, bc1_ref, pw_ref, wc2_ref, bc2_ref, out_ref):
  a = a0_ref[0] + a1_ref[0]
  deg = jnp.sum(deg_ref[...], axis=0)[:, None]
  agg = a / jnp.maximum(deg, 1.0)
  h = x_ref[...] @ w1a_ref[...] + agg @ w1b_ref[...] + b1_ref[...][None, :]
  h = jnp.maximum(h, 0.0)
  p1 = h @ wc1_ref[...] + bc1_ref[...][None, :]
  p1 = jnp.where(p1 >= 0, p1, pw_ref[...][None, :] * p1)
  out_ref[...] = p1 @ wc2_ref[...] + bc2_ref[...][None, :]


def _tc_mlp(x, partials, degs, w1a, w1b, b1, wc1, bc1, prelu_w, wc2, bc2):
  n, d = x.shape
  h = wc1.shape[0]
  c = wc2.shape[1]
  bn = 2048
  grid = -(-n // bn)

  full = lambda shape: pl.BlockSpec(shape, lambda i: (0,) * len(shape))
  return pl.pallas_call(
      _tc_mlp_body,
      grid=(grid,),
      in_specs=[
          pl.BlockSpec((bn, d), lambda i: (i, 0)),
          pl.BlockSpec((1, bn, d), lambda i: (0, i, 0)),
          pl.BlockSpec((1, bn, d), lambda i: (1, i, 0)),
          pl.BlockSpec((NW, bn), lambda i: (0, i)),
          full((d, h)), full((d, h)), full((h,)),
          full((h, h)), full((h,)), full((h,)),
          full((h, c)), full((c,)),
      ],
      out_specs=pl.BlockSpec((bn, c), lambda i: (i, 0)),
      out_shape=jax.ShapeDtypeStruct((n, c), jnp.float32),
  )(x, partials, partials, degs, w1a, w1b, b1, wc1, bc1, prelu_w, wc2, bc2)


def kernel(x, edge_index, labels, W1, b1, Wc1, bc1, prelu_w, Wc2, bc2):
  n, d = x.shape
  e = edge_index.shape[1]
  nit = (-(-e // (NW * CHUNK)) + 15) // 16 * 16
  per_tile = e // NW
  pad_per_tile = nit * CHUNK - per_tile
  n2 = n + NS
  rpt2 = (n2 // NS) // 8 * 8
  zeros = jnp.zeros((n2 - (NS - 1) * rpt2, d), jnp.float32)
  lane = jnp.arange(NW, dtype=edge_index.dtype)
  ghost_dst = jnp.broadcast_to((n + lane % NS)[:, None], (NW, pad_per_tile))
  ghost_src = jnp.broadcast_to(((lane * (n // NW)) % n)[:, None],
                               (NW, pad_per_tile))
  src = jnp.concatenate(
      [edge_index[0].reshape(NW, per_tile), ghost_src], axis=1
  ).reshape(NW, nit, CHUNK)
  dst = jnp.concatenate(
      [edge_index[1].reshape(NW, per_tile), ghost_dst], axis=1
  ).reshape(NW, nit, CHUNK)
  partials, degs = _sc_segsum(x, src, dst, zeros, n2)
  return _tc_mlp(x, partials, degs, W1[:d], W1[d:], b1,
                 Wc1, bc1, prelu_w, Wc2, bc2)


# speedup vs baseline: 2.8702x; 1.0800x over previous
import functools

import jax
import jax.numpy as jnp
from jax import lax
from jax.experimental import pallas as pl
from jax.experimental.pallas import tpu as pltpu
from jax.experimental.pallas import tpu_sc as plsc

NC, NS = 2, 16
NW = NC * NS
CHUNK = 128
STAGE = 16
LANES = 16


def _sc_segsum(x, src, dst, zeros, n2):
  n, d = x.shape
  nit = src.shape[1]
  ndeg = (n2 + 2 * LANES - 1) // LANES * LANES
  rpt = (n // NS) // 8 * 8
  tail = n - (NS - 1) * rpt
  rpt2 = (n2 // NS) // 8 * 8
  tail2 = n2 - (NS - 1) * rpt2

  mesh = plsc.VectorSubcoreMesh(core_axis_name="c", subcore_axis_name="s")

  @functools.partial(
      pl.kernel,
      out_type=(jax.ShapeDtypeStruct((NC, n, d), jnp.float32),
                jax.ShapeDtypeStruct((NW, ndeg), jnp.float32)),
      mesh=mesh,
      compiler_params=pltpu.CompilerParams(needs_layout_passes=False),
      scratch_types=[
          pltpu.VMEM_SHARED((n2, d), jnp.float32),
          pltpu.VMEM((STAGE, CHUNK), jnp.int32),
          pltpu.VMEM((STAGE, CHUNK), jnp.int32),
          pltpu.VMEM((2, CHUNK, d), jnp.float32),
          pltpu.VMEM((ndeg,), jnp.float32),
          pltpu.SemaphoreType.DMA,
          pltpu.SemaphoreType.DMA,
          pltpu.SemaphoreType.DMA,
          pltpu.SemaphoreType.DMA,
      ],
  )
  def seg_kernel(x_hbm, src_hbm, dst_hbm, zeros_hbm, zdeg_hbm, out_hbm,
                 deg_hbm, acc, src_v, dst_v, rows_v, deg_v,
                 gsem0, gsem1, ssem0, ssem1):
    cid = lax.axis_index("c")
    sid = lax.axis_index("s")
    wid = cid * NS + sid
    gsems = (gsem0, gsem1)
    ssems = (ssem0, ssem1)

    z0 = sid * rpt2
    pltpu.sync_copy(zeros_hbm.at[pl.ds(0, rpt2)], acc.at[pl.ds(z0, rpt2)])

    @pl.when(sid == NS - 1)
    def _zero_tail():
      t0 = NS * rpt2
      pltpu.sync_copy(zeros_hbm.at[pl.ds(0, tail2 - rpt2)],
                      acc.at[pl.ds(t0, tail2 - rpt2)])

    pltpu.sync_copy(zdeg_hbm, deg_v)
    plsc.subcore_barrier()

    def gather_start(l, p):
      pltpu.async_copy(x_hbm.at[src_v.at[l]], rows_v.at[p], gsems[p])

    def gather_wait(l, p):
      pltpu.make_async_copy(x_hbm.at[src_v.at[l]], rows_v.at[p],
                            gsems[p]).wait()

    def scatter_start(l, p):
      pltpu.async_copy(rows_v.at[p], acc.at[dst_v.at[l]], ssems[p], add=True)

    def scatter_wait(l, p):
      pltpu.make_async_copy(rows_v.at[p], acc.at[dst_v.at[l]],
                            ssems[p]).wait()

    def degrees(l):
      for j in range(CHUNK // LANES):
        dvec = dst_v[l, pl.ds(j * LANES, LANES)]
        cnt, last = plsc.scan_count(dvec)
        plsc.addupdate_scatter(deg_v, [dvec], cnt.astype(jnp.float32),
                               mask=last)

    def step_mid(l, p):
      scatter_wait(l - 1, 1 - p)
      gather_start(l + 1, 1 - p)
      gather_wait(l, p)
      scatter_start(l, p)
      degrees(l)

    for base in range(0, nit, STAGE):
      pltpu.sync_copy(src_hbm.at[wid, pl.ds(base, STAGE)], src_v)
      pltpu.sync_copy(dst_hbm.at[wid, pl.ds(base, STAGE)], dst_v)
      gather_start(0, 0)
      gather_wait(0, 0)
      gather_start(1, 1)
      scatter_start(0, 0)
      degrees(0)

      def body(o, _):
        step_mid(2 * o + 1, 1)
        step_mid(2 * o + 2, 0)
        return _

      lax.fori_loop(0, STAGE // 2 - 1, body, 0, unroll=False)
      scatter_wait(STAGE - 2, 0)
      gather_wait(STAGE - 1, 1)
      scatter_start(STAGE - 1, 1)
      degrees(STAGE - 1)
      scatter_wait(STAGE - 1, 1)
    plsc.subcore_barrier()

    r0 = sid * rpt
    pltpu.sync_copy(acc.at[pl.ds(r0, rpt)],
                    out_hbm.at[cid, pl.ds(r0, rpt)])

    @pl.when(sid == NS - 1)
    def _write_tail():
      t0 = NS * rpt
      pltpu.sync_copy(acc.at[pl.ds(t0, tail - rpt)],
                      out_hbm.at[cid, pl.ds(t0, tail - rpt)])

    pltpu.sync_copy(deg_v, deg_hbm.at[wid])

  return seg_kernel(x, src, dst, zeros, jnp.zeros((ndeg,), jnp.float32))


def _tc_mlp_body(x_ref, a0_ref, a1_ref, deg_ref, w1a_ref, w1b_ref, b1_ref,
                 wc1_ref, bc1_ref, pw_ref, wc2_ref, bc2_ref, out_ref):
  a = a0_ref[0] + a1_ref[0]
  deg = jnp.sum(deg_ref[...], axis=0)[:, None]
  agg = a / jnp.maximum(deg, 1.0)
  h = x_ref[...] @ w1a_ref[...] + agg @ w1b_ref[...] + b1_ref[...][None, :]
  h = jnp.maximum(h, 0.0)
  p1 = h @ wc1_ref[...] + bc1_ref[...][None, :]
  p1 = jnp.where(p1 >= 0, p1, pw_ref[...][None, :] * p1)
  out_ref[...] = p1 @ wc2_ref[...] + bc2_ref[...][None, :]


def _tc_mlp(x, partials, degs, w1a, w1b, b1, wc1, bc1, prelu_w, wc2, bc2):
  n, d = x.shape
  h = wc1.shape[0]
  c = wc2.shape[1]
  bn = 2048
  grid = -(-n // bn)

  full = lambda shape: pl.BlockSpec(shape, lambda i: (0,) * len(shape))
  return pl.pallas_call(
      _tc_mlp_body,
      grid=(grid,),
      in_specs=[
          pl.BlockSpec((bn, d), lambda i: (i, 0)),
          pl.BlockSpec((1, bn, d), lambda i: (0, i, 0)),
          pl.BlockSpec((1, bn, d), lambda i: (1, i, 0)),
          pl.BlockSpec((NW, bn), lambda i: (0, i)),
          full((d, h)), full((d, h)), full((h,)),
          full((h, h)), full((h,)), full((h,)),
          full((h, c)), full((c,)),
      ],
      out_specs=pl.BlockSpec((bn, c), lambda i: (i, 0)),
      out_shape=jax.ShapeDtypeStruct((n, c), jnp.float32),
  )(x, partials, partials, degs, w1a, w1b, b1, wc1, bc1, prelu_w, wc2, bc2)


def kernel(x, edge_index, labels, W1, b1, Wc1, bc1, prelu_w, Wc2, bc2):
  n, d = x.shape
  e = edge_index.shape[1]
  nit = (-(-e // (NW * CHUNK)) + STAGE - 1) // STAGE * STAGE
  per_tile = e // NW
  pad_per_tile = nit * CHUNK - per_tile
  n2 = n + NS
  rpt2 = (n2 // NS) // 8 * 8
  zeros = jnp.zeros((n2 - (NS - 1) * rpt2, d), jnp.float32)
  lane = jnp.arange(NW, dtype=edge_index.dtype)
  ghost_dst = jnp.broadcast_to((n + lane % NS)[:, None], (NW, pad_per_tile))
  ghost_src = jnp.broadcast_to(((lane * (n // NW)) % n)[:, None],
                               (NW, pad_per_tile))
  src = jnp.concatenate(
      [edge_index[0].reshape(NW, per_tile), ghost_src], axis=1
  ).reshape(NW, nit, CHUNK)
  dst = jnp.concatenate(
      [edge_index[1].reshape(NW, per_tile), ghost_dst], axis=1
  ).reshape(NW, nit, CHUNK)
  partials, degs = _sc_segsum(x, src, dst, zeros, n2)
  return _tc_mlp(x, partials, degs, W1[:d], W1[d:], b1,
                 Wc1, bc1, prelu_w, Wc2, bc2)
